# baseline (speedup 1.0000x reference)
import jax
import jax.numpy as jnp
from jax import lax
from jax.experimental import pallas as pl
from jax.experimental.pallas import tpu as pltpu
from jax.experimental.pallas import tpu_sc as plsc

B = 16384
N_CAT = 26
N_CONT = 13
D = 64

NC = 2
NS = 16
NW = NC * NS
BPW = B // NW
CB = 256
CPJ = BPW // CB
NCH = N_CAT * CPJ


def _cat_sc_kernel(idxT_hbm, table_hbm, bias_hbm, out_hbm,
                   idx_all, r0, r1, r2, r3, bias_v,
                   g0, g1, g2, g3, w0, w1, w2, w3):
    wid = lax.axis_index("s") * NC + lax.axis_index("c")
    wb0 = wid * BPW
    rbufs = (r0, r1, r2, r3)
    gsems = (g0, g1, g2, g3)
    wsems = (w0, w1, w2, w3)

    pltpu.sync_copy(idxT_hbm.at[:, pl.ds(wb0, BPW)], idx_all)
    pltpu.sync_copy(bias_hbm, bias_v)

    def fire(c, k):
        j = c // CPJ
        h = c % CPJ
        pltpu.make_async_copy(
            table_hbm.at[idx_all.at[j, pl.ds(h * CB, CB)]],
            rbufs[k], gsems[k]).start()

    def wait_gather(c, k):
        j = c // CPJ
        h = c % CPJ
        pltpu.make_async_copy(
            table_hbm.at[idx_all.at[j, pl.ds(h * CB, CB)]],
            rbufs[k], gsems[k]).wait()

    def write(c, k):
        j = c // CPJ
        b0 = wb0 + (c % CPJ) * CB
        pltpu.make_async_copy(rbufs[k],
                              out_hbm.at[j, pl.ds(b0, CB)],
                              wsems[k]).start()

    def wait_write(c, k):
        j = c // CPJ
        b0 = wb0 + (c % CPJ) * CB
        pltpu.make_async_copy(rbufs[k],
                              out_hbm.at[j, pl.ds(b0, CB)],
                              wsems[k]).wait()

    def bias_pass(c, k):
        j = c // CPJ
        rows = rbufs[k]
        b4 = [bias_v[j, pl.ds(v * 16, 16)] for v in range(D // 16)]

        @plsc.parallel_loop(0, CB // 8, unroll=2)
        def _bp(r8, rows=rows, b4=b4):
            for u in range(8):
                r = r8 * 8 + u
                for v in range(D // 16):
                    sl = pl.ds(v * 16, 16)
                    rows[r, sl] = rows[r, sl] + b4[v]

    fire(0, 0)
    fire(1, 1)

    def body(i, carry):
        c0 = 4 * i
        for u in range(4):
            c = c0 + u
            k = u
            wait_gather(c, k)
            bias_pass(c, k)
            write(c, k)
            kn = (u + 2) % 4

            @pl.when(c >= 2)
            def _(c=c, kn=kn):
                wait_write(c - 2, kn)

            @pl.when(c + 2 < NCH)
            def _(c=c, kn=kn):
                fire(c + 2, kn)
        return carry

    lax.fori_loop(0, NCH // 4, body, 0)
    wait_write(NCH - 2, 2)
    wait_write(NCH - 1, 3)


def _cat_call(idxT, table, cat_bias):
    mesh = plsc.VectorSubcoreMesh(core_axis_name="c", subcore_axis_name="s",
                                  num_cores=NC, num_subcores=NS)
    return pl.kernel(
        _cat_sc_kernel,
        out_type=jax.ShapeDtypeStruct((N_CAT, B, D), jnp.float32),
        mesh=mesh,
        scratch_types=[
            pltpu.VMEM((N_CAT, BPW), jnp.int32),
            pltpu.VMEM((CB, D), jnp.float32),
            pltpu.VMEM((CB, D), jnp.float32),
            pltpu.VMEM((CB, D), jnp.float32),
            pltpu.VMEM((CB, D), jnp.float32),
            pltpu.VMEM((N_CAT, D), jnp.float32),
            pltpu.SemaphoreType.DMA,
            pltpu.SemaphoreType.DMA,
            pltpu.SemaphoreType.DMA,
            pltpu.SemaphoreType.DMA,
            pltpu.SemaphoreType.DMA,
            pltpu.SemaphoreType.DMA,
            pltpu.SemaphoreType.DMA,
            pltpu.SemaphoreType.DMA,
        ],
        compiler_params=pltpu.CompilerParams(use_tc_tiling_on_sc=False,
                                             needs_layout_passes=False),
    )(idxT, table, cat_bias)


XPB = 512


def _xpose_tc_kernel(x_ref, o_ref):
    x = x_ref[0]
    n = XPB // 2
    eye = (lax.broadcasted_iota(jnp.int32, (n, n), 0)
           == lax.broadcasted_iota(jnp.int32, (n, n), 1)
           ).astype(jnp.float32)
    xt = lax.dot_general(x, eye, (((0,), (0,)), ((), ())),
                         preferred_element_type=jnp.float32)
    o_ref[0, :, :n] = xt[:D]
    o_ref[0, :, n:] = xt[D:]


def _xpose_call(cat_v):
    grid = (N_CAT, B // XPB)
    return pl.pallas_call(
        _xpose_tc_kernel,
        grid=grid,
        in_specs=[pl.BlockSpec((1, XPB // 2, 128), lambda j, w: (j, w, 0))],
        out_specs=pl.BlockSpec((1, D, XPB), lambda j, w: (j, 0, w)),
        out_shape=jax.ShapeDtypeStruct((N_CAT, D, B), jnp.float32),
    )(cat_v)


BLK = 2048


def _cont_tc_kernel(xcT_ref, g_ref, b_ref, wT_ref, bT_ref, o_ref):
    xc = xcT_ref[...]
    mu = jnp.mean(xc, axis=0, keepdims=True)
    var = jnp.mean((xc - mu) ** 2, axis=0, keepdims=True)
    xcn = (xc - mu) * lax.rsqrt(var + 1e-5)
    xcn = xcn * g_ref[...] + b_ref[...]
    for j in range(N_CONT):
        o_ref[j, :, :] = (xcn[j:j + 1, :] * wT_ref[:, j:j + 1]
                          + bT_ref[:, j:j + 1])


def _cont_call(xcT, ln_gamma, ln_beta, cont_wT, cont_bT):
    grid = (B // BLK,)
    return pl.pallas_call(
        _cont_tc_kernel,
        grid=grid,
        in_specs=[
            pl.BlockSpec((N_CONT, BLK), lambda i: (0, i)),
            pl.BlockSpec((N_CONT, 1), lambda i: (0, 0)),
            pl.BlockSpec((N_CONT, 1), lambda i: (0, 0)),
            pl.BlockSpec((D, N_CONT), lambda i: (0, 0)),
            pl.BlockSpec((D, N_CONT), lambda i: (0, 0)),
        ],
        out_specs=pl.BlockSpec((N_CONT, D, BLK), lambda i: (0, 0, i)),
        out_shape=jax.ShapeDtypeStruct((N_CONT, D, B), jnp.float32),
    )(xcT, ln_gamma.reshape(N_CONT, 1), ln_beta.reshape(N_CONT, 1),
      cont_wT, cont_bT)


def kernel(X, table, cat_bias, ln_gamma, ln_beta, cont_w, cont_b):
    XT = X.T
    idxT = XT[:N_CAT]
    xcT = XT[N_CAT:].astype(jnp.float32)
    idxp = (idxT.reshape(N_CAT, NW, 2, XPB // 2)
            .transpose(0, 1, 3, 2).reshape(N_CAT, B))
    cat_3d = _cat_call(idxp, table, cat_bias)
    cat_v = cat_3d.reshape(N_CAT, B * D // 128, 128)
    cat_T = _xpose_call(cat_v)
    cont_T = _cont_call(xcT, ln_gamma, ln_beta,
                        cont_w.T, cont_b.T)
    x_cat = jnp.transpose(cat_T, (2, 0, 1))
    x_cont = jnp.transpose(cont_T, (2, 0, 1))
    return x_cat, x_cont


# speedup vs baseline: 1.5046x; 1.5046x over previous
import jax
import jax.numpy as jnp
from jax import lax
from jax.experimental import pallas as pl
from jax.experimental.pallas import tpu as pltpu
from jax.experimental.pallas import tpu_sc as plsc

B = 16384
N_CAT = 26
N_CONT = 13
D = 64

NC = 2
NS = 16
NW = NC * NS
BPW = B // NW
CB = 256
CPJ = BPW // CB
NCH = N_CAT * CPJ


def _cat_sc_kernel(idxT_hbm, table_hbm, bias_hbm, out_hbm,
                   idx_all, r0, r1, r2, r3, bias_v,
                   g0, g1, g2, g3, w0, w1, w2, w3):
    wid = lax.axis_index("s") * NC + lax.axis_index("c")
    wb0 = wid * BPW
    rbufs = (r0, r1, r2, r3)
    gsems = (g0, g1, g2, g3)
    wsems = (w0, w1, w2, w3)

    pltpu.sync_copy(idxT_hbm.at[:, pl.ds(wb0, BPW)], idx_all)
    pltpu.sync_copy(bias_hbm, bias_v)

    def fire(c, k):
        j = c // CPJ
        h = c % CPJ
        pltpu.make_async_copy(
            table_hbm.at[idx_all.at[j, pl.ds(h * CB, CB)]],
            rbufs[k], gsems[k]).start()

    def wait_gather(c, k):
        j = c // CPJ
        h = c % CPJ
        pltpu.make_async_copy(
            table_hbm.at[idx_all.at[j, pl.ds(h * CB, CB)]],
            rbufs[k], gsems[k]).wait()

    def write(c, k):
        j = c // CPJ
        b0 = wb0 + (c % CPJ) * CB
        pltpu.make_async_copy(rbufs[k],
                              out_hbm.at[j, pl.ds(b0, CB)],
                              wsems[k]).start()

    def wait_write(c, k):
        j = c // CPJ
        b0 = wb0 + (c % CPJ) * CB
        pltpu.make_async_copy(rbufs[k],
                              out_hbm.at[j, pl.ds(b0, CB)],
                              wsems[k]).wait()

    def bias_pass(c, k):
        j = c // CPJ
        rows = rbufs[k]
        b4 = [bias_v[j, pl.ds(v * 16, 16)] for v in range(D // 16)]

        @plsc.parallel_loop(0, CB // 8, unroll=2)
        def _bp(r8, rows=rows, b4=b4):
            for u in range(8):
                r = r8 * 8 + u
                for v in range(D // 16):
                    sl = pl.ds(v * 16, 16)
                    rows[r, sl] = rows[r, sl] + b4[v]

    fire(0, 0)
    fire(1, 1)

    def body(i, carry):
        c0 = 4 * i
        for u in range(4):
            c = c0 + u
            k = u
            wait_gather(c, k)
            bias_pass(c, k)
            write(c, k)
            kn = (u + 2) % 4

            @pl.when(c >= 2)
            def _(c=c, kn=kn):
                wait_write(c - 2, kn)

            @pl.when(c + 2 < NCH)
            def _(c=c, kn=kn):
                fire(c + 2, kn)
        return carry

    lax.fori_loop(0, NCH // 4, body, 0)
    wait_write(NCH - 2, 2)
    wait_write(NCH - 1, 3)


def _cat_call(idxT, table, cat_bias):
    mesh = plsc.VectorSubcoreMesh(core_axis_name="c", subcore_axis_name="s",
                                  num_cores=NC, num_subcores=NS)
    return pl.kernel(
        _cat_sc_kernel,
        out_type=jax.ShapeDtypeStruct((N_CAT, B, D), jnp.float32),
        mesh=mesh,
        scratch_types=[
            pltpu.VMEM((N_CAT, BPW), jnp.int32),
            pltpu.VMEM((CB, D), jnp.float32),
            pltpu.VMEM((CB, D), jnp.float32),
            pltpu.VMEM((CB, D), jnp.float32),
            pltpu.VMEM((CB, D), jnp.float32),
            pltpu.VMEM((N_CAT, D), jnp.float32),
            pltpu.SemaphoreType.DMA,
            pltpu.SemaphoreType.DMA,
            pltpu.SemaphoreType.DMA,
            pltpu.SemaphoreType.DMA,
            pltpu.SemaphoreType.DMA,
            pltpu.SemaphoreType.DMA,
            pltpu.SemaphoreType.DMA,
            pltpu.SemaphoreType.DMA,
        ],
        compiler_params=pltpu.CompilerParams(use_tc_tiling_on_sc=False,
                                             needs_layout_passes=False),
    )(idxT, table, cat_bias)


XPB = 512


def _xpose_tc_kernel(x_ref, o_ref):
    n = XPB // 2
    eye = (lax.broadcasted_iota(jnp.int32, (n, n), 0)
           == lax.broadcasted_iota(jnp.int32, (n, n), 1)
           ).astype(jnp.float32)
    for w in range(B // XPB):
        x = x_ref[0, pl.ds(w * n, n), :]
        xt = lax.dot_general(x, eye, (((0,), (0,)), ((), ())),
                             preferred_element_type=jnp.float32,
                             precision=lax.Precision.HIGHEST)
        o_ref[0, :, pl.ds(w * XPB, n)] = xt[:D]
        o_ref[0, :, pl.ds(w * XPB + n, n)] = xt[D:]


def _xpose_call(cat_v):
    grid = (N_CAT,)
    return pl.pallas_call(
        _xpose_tc_kernel,
        grid=grid,
        in_specs=[pl.BlockSpec((1, B * D // 128, 128), lambda j: (j, 0, 0))],
        out_specs=pl.BlockSpec((1, D, B), lambda j: (j, 0, 0)),
        out_shape=jax.ShapeDtypeStruct((N_CAT, D, B), jnp.float32),
    )(cat_v)


BLK = 2048


def _cont_tc_kernel(xcT_ref, g_ref, b_ref, wT_ref, bT_ref, o_ref):
    xc = xcT_ref[...]
    mu = jnp.mean(xc, axis=0, keepdims=True)
    var = jnp.mean((xc - mu) ** 2, axis=0, keepdims=True)
    xcn = (xc - mu) * lax.rsqrt(var + 1e-5)
    xcn = xcn * g_ref[...] + b_ref[...]
    for j in range(N_CONT):
        o_ref[j, :, :] = (xcn[j:j + 1, :] * wT_ref[:, j:j + 1]
                          + bT_ref[:, j:j + 1])


def _cont_call(xcT, ln_gamma, ln_beta, cont_wT, cont_bT):
    grid = (B // BLK,)
    return pl.pallas_call(
        _cont_tc_kernel,
        grid=grid,
        in_specs=[
            pl.BlockSpec((N_CONT, BLK), lambda i: (0, i)),
            pl.BlockSpec((N_CONT, 1), lambda i: (0, 0)),
            pl.BlockSpec((N_CONT, 1), lambda i: (0, 0)),
            pl.BlockSpec((D, N_CONT), lambda i: (0, 0)),
            pl.BlockSpec((D, N_CONT), lambda i: (0, 0)),
        ],
        out_specs=pl.BlockSpec((N_CONT, D, BLK), lambda i: (0, 0, i)),
        out_shape=jax.ShapeDtypeStruct((N_CONT, D, B), jnp.float32),
    )(xcT, ln_gamma.reshape(N_CONT, 1), ln_beta.reshape(N_CONT, 1),
      cont_wT, cont_bT)


def kernel(X, table, cat_bias, ln_gamma, ln_beta, cont_w, cont_b):
    XT = X.T
    idxT = XT[:N_CAT]
    xcT = XT[N_CAT:].astype(jnp.float32)
    idxp = (idxT.reshape(N_CAT, NW, 2, XPB // 2)
            .transpose(0, 1, 3, 2).reshape(N_CAT, B))
    cat_3d = _cat_call(idxp, table, cat_bias)
    cat_v = cat_3d.reshape(N_CAT, B * D // 128, 128)
    cat_T = _xpose_call(cat_v)
    cont_T = _cont_call(xcT, ln_gamma, ln_beta,
                        cont_w.T, cont_b.T)
    x_cat = jnp.transpose(cat_T, (2, 0, 1))
    x_cont = jnp.transpose(cont_T, (2, 0, 1))
    return x_cat, x_cont
